# baseline (speedup 1.0000x reference)
import functools

import jax
import jax.numpy as jnp
from jax import lax
from jax.experimental import pallas as pl
from jax.experimental.pallas import tpu as pltpu
from jax.experimental.pallas import tpu_sc as plsc

NN = 10000
EE = 320000
FD = 128
FH = FD // 2
NG = 64
NC = 2
NS = 16
NW = NC * NS
CH = 128
EPW = EE // NW
NCH = 80
EPAD = NCH * CH
PCH = 40
RPT = 632
NP = NS * RPT

BR = 1000
NBLK = NN // BR


def _sc_edge_agg_body(src_hbm, dst_hbm, table_hbm, out_hbm,
                      acc, sidx, didx, g0, s0):
    c = lax.axis_index("c")
    s = lax.axis_index("s")
    w = s * NC + c

    zero16 = jnp.zeros((16,), jnp.float32)

    def zrow(i, carry):
        for j in range(FD // 16):
            g0[i, pl.ds(j * 16, 16)] = zero16
        return carry

    lax.fori_loop(0, CH, zrow, 0)
    base = pl.multiple_of(s * RPT, 8)
    for k in range(RPT // CH):
        pltpu.sync_copy(g0, acc.at[pl.ds(base + k * CH, CH)])
    pltpu.sync_copy(g0.at[pl.ds(0, RPT % CH)],
                    acc.at[pl.ds(base + (RPT // CH) * CH, RPT % CH)])
    plsc.subcore_barrier()

    pltpu.sync_copy(src_hbm.at[w], sidx)
    pltpu.sync_copy(dst_hbm.at[w], didx)

    def chunk(j, carry):
        pltpu.async_copy(table_hbm.at[sidx.at[j]], g0, s0).wait()
        pltpu.sync_copy(g0, acc.at[didx.at[j]], add=True)
        return carry

    lax.fori_loop(0, NCH, chunk, 0)
    plsc.subcore_barrier()

    pltpu.sync_copy(acc.at[pl.ds(base, RPT)], out_hbm.at[c, pl.ds(base, RPT)])


@functools.cache
def _sc_edge_agg():
    mesh = plsc.VectorSubcoreMesh(core_axis_name="c", subcore_axis_name="s",
                                  num_cores=NC, num_subcores=NS)
    return pl.kernel(
        _sc_edge_agg_body,
        out_type=jax.ShapeDtypeStruct((NC, NP, FD), jnp.float32),
        mesh=mesh,
        scratch_types=[
            pltpu.VMEM_SHARED((NP, FD), jnp.float32),
            pltpu.VMEM((NCH, CH), jnp.int32),
            pltpu.VMEM((NCH, CH), jnp.int32),
            pltpu.VMEM((CH, FD), jnp.float32),
            pltpu.SemaphoreType.DMA,
        ],
    )


def _sc_deg_body(dst_hbm, out_hbm, acc, didx, obuf, sem):
    c = lax.axis_index("c")
    s = lax.axis_index("s")
    w = s * NC + c

    zero16 = jnp.zeros((16,), jnp.float32)

    def zrow(i, carry):
        obuf[i, pl.ds(0, 16)] = zero16
        return carry

    lax.fori_loop(0, CH, zrow, 0)
    base = pl.multiple_of(s * RPT, 8)
    for k in range(RPT // CH):
        pltpu.sync_copy(obuf, acc.at[pl.ds(base + k * CH, CH)])
    pltpu.sync_copy(obuf.at[pl.ds(0, RPT % CH)],
                    acc.at[pl.ds(base + (RPT // CH) * CH, RPT % CH)])

    one16 = jnp.ones((16,), jnp.float32)

    def orow(i, carry):
        obuf[i, pl.ds(0, 16)] = one16
        return carry

    lax.fori_loop(0, CH, orow, 0)
    pltpu.sync_copy(dst_hbm.at[w], didx)
    plsc.subcore_barrier()

    def chunk(j, carry):
        pltpu.sync_copy(obuf, acc.at[didx.at[j]], add=True)
        return carry

    lax.fori_loop(0, NCH, chunk, 0)
    plsc.subcore_barrier()
    pltpu.sync_copy(acc.at[pl.ds(base, RPT)], out_hbm.at[c, pl.ds(base, RPT)])


@functools.cache
def _sc_deg():
    mesh = plsc.VectorSubcoreMesh(core_axis_name="c", subcore_axis_name="s",
                                  num_cores=NC, num_subcores=NS)
    return pl.kernel(
        _sc_deg_body,
        out_type=jax.ShapeDtypeStruct((NC, NP, 16), jnp.float32),
        mesh=mesh,
        scratch_types=[
            pltpu.VMEM_SHARED((NP, 16), jnp.float32),
            pltpu.VMEM((NCH, CH), jnp.int32),
            pltpu.VMEM((CH, 16), jnp.float32),
            pltpu.SemaphoreType.DMA,
        ],
    )


def _tc_first_body(x_ref, w_ref, d0_ref, d1_ref, scaled_ref, dinv_ref):
    deg = d0_ref[...] + d1_ref[...] + 1.0
    dinv = lax.rsqrt(deg)
    y = jnp.dot(x_ref[...], w_ref[...], preferred_element_type=jnp.float32)
    scaled_ref[...] = dinv * y
    dinv_ref[...] = dinv


def _tc_first(x, w, d0, d1):
    return pl.pallas_call(
        _tc_first_body,
        grid=(NBLK,),
        in_specs=[
            pl.BlockSpec((BR, FD), lambda i: (i, 0)),
            pl.BlockSpec((FD, FD), lambda i: (0, 0)),
            pl.BlockSpec((BR, 1), lambda i: (i, 0)),
            pl.BlockSpec((BR, 1), lambda i: (i, 0)),
        ],
        out_specs=[
            pl.BlockSpec((BR, FD), lambda i: (i, 0)),
            pl.BlockSpec((BR, 1), lambda i: (i, 0)),
        ],
        out_shape=[
            jax.ShapeDtypeStruct((NN, FD), jnp.float32),
            jax.ShapeDtypeStruct((NN, 1), jnp.float32),
        ],
    )(x, w, d0, d1)


def _tc_mid_body(a0_ref, a1_ref, sp_ref, dinv_ref, b_ref, w_ref, out_ref):
    dinv = dinv_ref[...]
    h = dinv * (a0_ref[...] + a1_ref[...] + sp_ref[...]) + b_ref[...]
    h = jnp.maximum(h, 0.0)
    out_ref[...] = dinv * jnp.dot(h, w_ref[...],
                                  preferred_element_type=jnp.float32)


def _tc_mid(a0, a1, sp, dinv, b, w):
    return pl.pallas_call(
        _tc_mid_body,
        grid=(NBLK,),
        in_specs=[
            pl.BlockSpec((BR, FD), lambda i: (i, 0)),
            pl.BlockSpec((BR, FD), lambda i: (i, 0)),
            pl.BlockSpec((BR, FD), lambda i: (i, 0)),
            pl.BlockSpec((BR, 1), lambda i: (i, 0)),
            pl.BlockSpec((1, FD), lambda i: (0, 0)),
            pl.BlockSpec((FD, FD), lambda i: (0, 0)),
        ],
        out_specs=pl.BlockSpec((BR, FD), lambda i: (i, 0)),
        out_shape=jax.ShapeDtypeStruct((NN, FD), jnp.float32),
    )(a0, a1, sp, dinv, b, w)


def _tc_pool_body(a0_ref, a1_ref, sp_ref, dinv_ref, b_ref, batch_ref,
                  wl1_ref, bl1_ref, wl2_ref, bl2_ref, wl3_ref, bl3_ref,
                  out_ref, sum_s, max_s, cnt_s):
    i = pl.program_id(0)

    @pl.when(i == 0)
    def _():
        sum_s[...] = jnp.zeros_like(sum_s)
        cnt_s[...] = jnp.zeros_like(cnt_s)
        max_s[...] = jnp.full_like(max_s, -1e30)

    dinv = dinv_ref[...]
    h = dinv * (a0_ref[...] + a1_ref[...] + sp_ref[...]) + b_ref[...]
    h = jnp.maximum(h, 0.0)

    bvec = batch_ref[...]
    bmin = jnp.min(bvec)
    bmax = jnp.max(bvec)
    for g in range(NG):
        @pl.when(jnp.logical_and(bmin <= g, g <= bmax))
        def _(g=g):
            m = bvec == g
            hm = jnp.where(m, h, 0.0)
            sum_s[g:g + 1, :] = sum_s[g:g + 1, :] + jnp.sum(
                hm, axis=0, keepdims=True)
            cnt = jnp.sum(jnp.where(m, 1.0, 0.0), axis=0, keepdims=True)
            cnt_s[g:g + 1, :] = cnt_s[g:g + 1, :] + cnt
            mx = jnp.max(jnp.where(m, h, -1e30), axis=0, keepdims=True)
            max_s[g:g + 1, :] = jnp.maximum(max_s[g:g + 1, :], mx)

    @pl.when(i == NBLK - 1)
    def _():
        cnt = cnt_s[...]
        sm = sum_s[...]
        mean = sm / jnp.maximum(cnt, 1.0)
        mx = jnp.where(cnt > 0.0, max_s[...], 0.0)
        hg = jnp.concatenate([mean, mx, sm], axis=1)
        z = jnp.dot(hg, wl1_ref[...],
                    preferred_element_type=jnp.float32) + bl1_ref[...]
        z = jnp.maximum(z, 0.0)
        z = jnp.dot(z, wl2_ref[...],
                    preferred_element_type=jnp.float32) + bl2_ref[...]
        z = jnp.maximum(z, 0.0)
        z = jnp.dot(z, wl3_ref[...],
                    preferred_element_type=jnp.float32) + bl3_ref[...]
        out_ref[...] = 1.0 / (1.0 + jnp.exp(-z))


def _tc_pool(a0, a1, sp, dinv, b, batch2d, wl1, bl1, wl2, bl2, wl3p, bl3):
    return pl.pallas_call(
        _tc_pool_body,
        grid=(NBLK,),
        in_specs=[
            pl.BlockSpec((BR, FD), lambda i: (i, 0)),
            pl.BlockSpec((BR, FD), lambda i: (i, 0)),
            pl.BlockSpec((BR, FD), lambda i: (i, 0)),
            pl.BlockSpec((BR, 1), lambda i: (i, 0)),
            pl.BlockSpec((1, FD), lambda i: (0, 0)),
            pl.BlockSpec((BR, 1), lambda i: (i, 0)),
            pl.BlockSpec((3 * FD, 3 * FD), lambda i: (0, 0)),
            pl.BlockSpec((1, 3 * FD), lambda i: (0, 0)),
            pl.BlockSpec((3 * FD, FD), lambda i: (0, 0)),
            pl.BlockSpec((1, FD), lambda i: (0, 0)),
            pl.BlockSpec((FD, FD), lambda i: (0, 0)),
            pl.BlockSpec((1, 1), lambda i: (0, 0)),
        ],
        out_specs=pl.BlockSpec((NG, FD), lambda i: (0, 0)),
        out_shape=jax.ShapeDtypeStruct((NG, FD), jnp.float32),
        scratch_shapes=[
            pltpu.VMEM((NG, FD), jnp.float32),
            pltpu.VMEM((NG, FD), jnp.float32),
            pltpu.VMEM((NG, FD), jnp.float32),
        ],
    )(a0, a1, sp, dinv, b, batch2d, wl1, bl1, wl2, bl2, wl3p, bl3)


def kernel(x, edge_index, batch, Wg0, bg0, Wg1, bg1,
           Wl1, bl1, Wl2, bl2, Wl3, bl3):
    src = edge_index[0]
    dst = edge_index[1]
    srcp = jnp.pad(src.reshape(NW, EPW),
                   ((0, 0), (0, EPAD - EPW))).reshape(NW, NCH, CH)
    dstp = jnp.pad(dst.reshape(NW, EPW), ((0, 0), (0, EPAD - EPW)),
                   constant_values=NN).reshape(NW, NCH, CH)

    degraw = _sc_deg()(dstp)
    d0 = degraw[0, :, 0:1]
    d1 = degraw[1, :, 0:1]

    scaled0, dinv = _tc_first(x, Wg0, d0, d1)
    agg = _sc_edge_agg()(srcp, dstp, scaled0)
    scaled1 = _tc_mid(agg[0], agg[1], scaled0, dinv, bg0.reshape(1, FD), Wg1)
    agg = _sc_edge_agg()(srcp, dstp, scaled1)
    scaled2 = _tc_mid(agg[0], agg[1], scaled1, dinv, bg1.reshape(1, FD), Wg1)
    agg = _sc_edge_agg()(srcp, dstp, scaled2)

    wl3p = jnp.pad(Wl3, ((0, 0), (0, FD - 1)))
    outp = _tc_pool(agg[0], agg[1], scaled2, dinv, bg1.reshape(1, FD),
                    batch.reshape(NN, 1).astype(jnp.int32),
                    Wl1, bl1.reshape(1, 3 * FD), Wl2, bl2.reshape(1, FD),
                    wl3p, bl3.reshape(1, 1))
    return outp[:, 0:1]


# speedup vs baseline: 2.2783x; 2.2783x over previous
import functools

import jax
import jax.numpy as jnp
from jax import lax
from jax.experimental import pallas as pl
from jax.experimental.pallas import tpu as pltpu
from jax.experimental.pallas import tpu_sc as plsc

NN = 10000
EE = 320000
FD = 128
FH = FD // 2
NG = 64
NC = 2
NS = 16
NW = NC * NS
CH = 128
EPW = EE // NW
NCH = 79
EPAD = NCH * CH
RPT = 632
NP = NS * RPT

BR = 1000
NBLK = NN // BR


def _sc_edge_agg_body(src_hbm, dst_hbm, table_hbm, out_hbm,
                      acc, sidx, didx, g0, s0):
    c = lax.axis_index("c")
    s = lax.axis_index("s")
    w = s * NC + c

    zero16 = jnp.zeros((16,), jnp.float32)

    def zrow(i, carry):
        for j in range(FD // 16):
            g0[i, pl.ds(j * 16, 16)] = zero16
        return carry

    lax.fori_loop(0, CH, zrow, 0)
    base = pl.multiple_of(s * RPT, 8)
    for k in range(RPT // CH):
        pltpu.sync_copy(g0, acc.at[pl.ds(base + k * CH, CH)])
    pltpu.sync_copy(g0.at[pl.ds(0, RPT % CH)],
                    acc.at[pl.ds(base + (RPT // CH) * CH, RPT % CH)])
    plsc.subcore_barrier()

    pltpu.sync_copy(src_hbm.at[w], sidx)
    pltpu.sync_copy(dst_hbm.at[w], didx)

    def chunk(j, carry):
        pltpu.async_copy(table_hbm.at[sidx.at[j]], g0, s0).wait()
        pltpu.sync_copy(g0, acc.at[didx.at[j]], add=True)
        return carry

    lax.fori_loop(0, NCH, chunk, 0)
    plsc.subcore_barrier()

    pltpu.sync_copy(acc.at[pl.ds(base, RPT)], out_hbm.at[c, pl.ds(base, RPT)])


@functools.cache
def _sc_edge_agg():
    mesh = plsc.VectorSubcoreMesh(core_axis_name="c", subcore_axis_name="s",
                                  num_cores=NC, num_subcores=NS)
    return pl.kernel(
        _sc_edge_agg_body,
        out_type=jax.ShapeDtypeStruct((NC, NP, FD), jnp.float32),
        mesh=mesh,
        scratch_types=[
            pltpu.VMEM_SHARED((NP, FD), jnp.float32),
            pltpu.VMEM((NCH, CH), jnp.int32),
            pltpu.VMEM((NCH, CH), jnp.int32),
            pltpu.VMEM((CH, FD), jnp.float32),
            pltpu.SemaphoreType.DMA,
        ],
    )


def _sc_deg_body(dst_hbm, out_hbm, acc, didx, obuf, sem):
    c = lax.axis_index("c")
    s = lax.axis_index("s")
    w = s * NC + c

    zero16 = jnp.zeros((16,), jnp.float32)

    def zrow(i, carry):
        obuf[i, pl.ds(0, 16)] = zero16
        return carry

    lax.fori_loop(0, CH, zrow, 0)
    base = pl.multiple_of(s * RPT, 8)
    for k in range(RPT // CH):
        pltpu.sync_copy(obuf, acc.at[pl.ds(base + k * CH, CH)])
    pltpu.sync_copy(obuf.at[pl.ds(0, RPT % CH)],
                    acc.at[pl.ds(base + (RPT // CH) * CH, RPT % CH)])

    one16 = jnp.ones((16,), jnp.float32)

    def orow(i, carry):
        obuf[i, pl.ds(0, 16)] = one16
        return carry

    lax.fori_loop(0, CH, orow, 0)
    pltpu.sync_copy(dst_hbm.at[w], didx)
    plsc.subcore_barrier()

    def chunk(j, carry):
        pltpu.sync_copy(obuf, acc.at[didx.at[j]], add=True)
        return carry

    lax.fori_loop(0, NCH, chunk, 0)
    plsc.subcore_barrier()
    pltpu.sync_copy(acc.at[pl.ds(base, RPT)], out_hbm.at[c, pl.ds(base, RPT)])


@functools.cache
def _sc_deg():
    mesh = plsc.VectorSubcoreMesh(core_axis_name="c", subcore_axis_name="s",
                                  num_cores=NC, num_subcores=NS)
    return pl.kernel(
        _sc_deg_body,
        out_type=jax.ShapeDtypeStruct((NC, NP, 16), jnp.float32),
        mesh=mesh,
        scratch_types=[
            pltpu.VMEM_SHARED((NP, 16), jnp.float32),
            pltpu.VMEM((NCH, CH), jnp.int32),
            pltpu.VMEM((CH, 16), jnp.float32),
            pltpu.SemaphoreType.DMA,
        ],
    )


def _tc_first_body(x_ref, w_ref, d0_ref, d1_ref, scaled_ref, dinv_ref):
    deg = d0_ref[...] + d1_ref[...] + 1.0
    dinv = lax.rsqrt(deg)
    y = jnp.dot(x_ref[...], w_ref[...], preferred_element_type=jnp.float32)
    scaled_ref[...] = dinv * y
    dinv_ref[...] = dinv


def _tc_first(x, w, d0, d1):
    return pl.pallas_call(
        _tc_first_body,
        grid=(NBLK,),
        in_specs=[
            pl.BlockSpec((BR, FD), lambda i: (i, 0)),
            pl.BlockSpec((FD, FD), lambda i: (0, 0)),
            pl.BlockSpec((BR, 1), lambda i: (i, 0)),
            pl.BlockSpec((BR, 1), lambda i: (i, 0)),
        ],
        out_specs=[
            pl.BlockSpec((BR, FD), lambda i: (i, 0)),
            pl.BlockSpec((BR, 1), lambda i: (i, 0)),
        ],
        out_shape=[
            jax.ShapeDtypeStruct((NN, FD), jnp.float32),
            jax.ShapeDtypeStruct((NN, 1), jnp.float32),
        ],
    )(x, w, d0, d1)


def _tc_mid_body(a0_ref, a1_ref, sp_ref, dinv_ref, b_ref, w_ref, out_ref):
    dinv = dinv_ref[...]
    h = dinv * (a0_ref[...] + a1_ref[...] + sp_ref[...]) + b_ref[...]
    h = jnp.maximum(h, 0.0)
    out_ref[...] = dinv * jnp.dot(h, w_ref[...],
                                  preferred_element_type=jnp.float32)


def _tc_mid(a0, a1, sp, dinv, b, w):
    return pl.pallas_call(
        _tc_mid_body,
        grid=(NBLK,),
        in_specs=[
            pl.BlockSpec((BR, FD), lambda i: (i, 0)),
            pl.BlockSpec((BR, FD), lambda i: (i, 0)),
            pl.BlockSpec((BR, FD), lambda i: (i, 0)),
            pl.BlockSpec((BR, 1), lambda i: (i, 0)),
            pl.BlockSpec((1, FD), lambda i: (0, 0)),
            pl.BlockSpec((FD, FD), lambda i: (0, 0)),
        ],
        out_specs=pl.BlockSpec((BR, FD), lambda i: (i, 0)),
        out_shape=jax.ShapeDtypeStruct((NN, FD), jnp.float32),
    )(a0, a1, sp, dinv, b, w)


def _tc_pool_body(a0_ref, a1_ref, sp_ref, dinv_ref, b_ref, batch_ref,
                  wl1_ref, bl1_ref, wl2_ref, bl2_ref, wl3_ref, bl3_ref,
                  out_ref, sum_s, max_s, cnt_s):
    i = pl.program_id(0)

    @pl.when(i == 0)
    def _():
        sum_s[...] = jnp.zeros_like(sum_s)
        cnt_s[...] = jnp.zeros_like(cnt_s)
        max_s[...] = jnp.full_like(max_s, -1e30)

    dinv = dinv_ref[...]
    h = dinv * (a0_ref[...] + a1_ref[...] + sp_ref[...]) + b_ref[...]
    h = jnp.maximum(h, 0.0)

    bvec = batch_ref[...]
    bmin = jnp.min(bvec)
    bmax = jnp.max(bvec)
    for g in range(NG):
        @pl.when(jnp.logical_and(bmin <= g, g <= bmax))
        def _(g=g):
            m = bvec == g
            hm = jnp.where(m, h, 0.0)
            sum_s[g:g + 1, :] = sum_s[g:g + 1, :] + jnp.sum(
                hm, axis=0, keepdims=True)
            cnt = jnp.sum(jnp.where(m, 1.0, 0.0), axis=0, keepdims=True)
            cnt_s[g:g + 1, :] = cnt_s[g:g + 1, :] + cnt
            mx = jnp.max(jnp.where(m, h, -1e30), axis=0, keepdims=True)
            max_s[g:g + 1, :] = jnp.maximum(max_s[g:g + 1, :], mx)

    @pl.when(i == NBLK - 1)
    def _():
        cnt = cnt_s[...]
        sm = sum_s[...]
        mean = sm / jnp.maximum(cnt, 1.0)
        mx = jnp.where(cnt > 0.0, max_s[...], 0.0)
        hg = jnp.concatenate([mean, mx, sm], axis=1)
        z = jnp.dot(hg, wl1_ref[...],
                    preferred_element_type=jnp.float32) + bl1_ref[...]
        z = jnp.maximum(z, 0.0)
        z = jnp.dot(z, wl2_ref[...],
                    preferred_element_type=jnp.float32) + bl2_ref[...]
        z = jnp.maximum(z, 0.0)
        z = jnp.dot(z, wl3_ref[...],
                    preferred_element_type=jnp.float32) + bl3_ref[...]
        out_ref[...] = 1.0 / (1.0 + jnp.exp(-z))


def _tc_pool(a0, a1, sp, dinv, b, batch2d, wl1, bl1, wl2, bl2, wl3p, bl3):
    return pl.pallas_call(
        _tc_pool_body,
        grid=(NBLK,),
        in_specs=[
            pl.BlockSpec((BR, FD), lambda i: (i, 0)),
            pl.BlockSpec((BR, FD), lambda i: (i, 0)),
            pl.BlockSpec((BR, FD), lambda i: (i, 0)),
            pl.BlockSpec((BR, 1), lambda i: (i, 0)),
            pl.BlockSpec((1, FD), lambda i: (0, 0)),
            pl.BlockSpec((BR, 1), lambda i: (i, 0)),
            pl.BlockSpec((3 * FD, 3 * FD), lambda i: (0, 0)),
            pl.BlockSpec((1, 3 * FD), lambda i: (0, 0)),
            pl.BlockSpec((3 * FD, FD), lambda i: (0, 0)),
            pl.BlockSpec((1, FD), lambda i: (0, 0)),
            pl.BlockSpec((FD, FD), lambda i: (0, 0)),
            pl.BlockSpec((1, 1), lambda i: (0, 0)),
        ],
        out_specs=pl.BlockSpec((NG, FD), lambda i: (0, 0)),
        out_shape=jax.ShapeDtypeStruct((NG, FD), jnp.float32),
        scratch_shapes=[
            pltpu.VMEM((NG, FD), jnp.float32),
            pltpu.VMEM((NG, FD), jnp.float32),
            pltpu.VMEM((NG, FD), jnp.float32),
        ],
    )(a0, a1, sp, dinv, b, batch2d, wl1, bl1, wl2, bl2, wl3p, bl3)


def kernel(x, edge_index, batch, Wg0, bg0, Wg1, bg1,
           Wl1, bl1, Wl2, bl2, Wl3, bl3):
    src = edge_index[0]
    dst = edge_index[1]
    npad = EPAD - EPW
    padd = NN + (jnp.arange(npad, dtype=jnp.int32) % (NP - NN))
    pads = jnp.arange(npad, dtype=jnp.int32) % NN
    srcp = jnp.concatenate(
        [src.reshape(NW, EPW),
         jnp.broadcast_to(pads, (NW, npad))], axis=1).reshape(NW, NCH, CH)
    dstp = jnp.concatenate(
        [dst.reshape(NW, EPW),
         jnp.broadcast_to(padd, (NW, npad))], axis=1).reshape(NW, NCH, CH)

    degraw = _sc_deg()(dstp)
    d0 = degraw[0, :, 0:1]
    d1 = degraw[1, :, 0:1]

    scaled0, dinv = _tc_first(x, Wg0, d0, d1)
    agg = _sc_edge_agg()(srcp, dstp, scaled0)
    scaled1 = _tc_mid(agg[0], agg[1], scaled0, dinv, bg0.reshape(1, FD), Wg1)
    agg = _sc_edge_agg()(srcp, dstp, scaled1)
    scaled2 = _tc_mid(agg[0], agg[1], scaled1, dinv, bg1.reshape(1, FD), Wg1)
    agg = _sc_edge_agg()(srcp, dstp, scaled2)

    wl3p = jnp.pad(Wl3, ((0, 0), (0, FD - 1)))
    outp = _tc_pool(agg[0], agg[1], scaled2, dinv, bg1.reshape(1, FD),
                    batch.reshape(NN, 1).astype(jnp.int32),
                    Wl1, bl1.reshape(1, 3 * FD), Wl2, bl2.reshape(1, FD),
                    wl3p, bl3.reshape(1, 1))
    return outp[:, 0:1]


# speedup vs baseline: 3.2124x; 1.4100x over previous
import functools

import jax
import jax.numpy as jnp
from jax import lax
from jax.experimental import pallas as pl
from jax.experimental.pallas import tpu as pltpu
from jax.experimental.pallas import tpu_sc as plsc

NN = 10000
EE = 320000
FD = 128
FH = FD // 2
NG = 64
NC = 2
NS = 16
NW = NC * NS
CH = 128
EPW = EE // NW
NCH = 79
EPAD = NCH * CH
PCH = 40
RPT = 632
NP = NS * RPT

BR = 1000
NBLK = NN // BR


def _run_block(table_hbm, acc, sbuf, dbuf, g0, g1, s0, s1, nch):
    pltpu.async_copy(table_hbm.at[sbuf.at[0]], g0, s0)

    def pair(jj, carry):
        j = jj * 2
        pltpu.async_copy(table_hbm.at[sbuf.at[j + 1]], g1, s1)
        pltpu.make_async_copy(table_hbm.at[pl.ds(0, CH)], g0, s0).wait()
        pltpu.sync_copy(g0, acc.at[dbuf.at[j]], add=True)
        pltpu.async_copy(table_hbm.at[sbuf.at[j + 2]], g0, s0)
        pltpu.make_async_copy(table_hbm.at[pl.ds(0, CH)], g1, s1).wait()
        pltpu.sync_copy(g1, acc.at[dbuf.at[j + 1]], add=True)
        return carry

    lax.fori_loop(0, (nch - 1) // 2, pair, 0)
    if nch % 2 == 0:
        pltpu.async_copy(table_hbm.at[sbuf.at[nch - 1]], g1, s1)
        pltpu.make_async_copy(table_hbm.at[pl.ds(0, CH)], g0, s0).wait()
        pltpu.sync_copy(g0, acc.at[dbuf.at[nch - 2]], add=True)
        pltpu.make_async_copy(table_hbm.at[pl.ds(0, CH)], g1, s1).wait()
        pltpu.sync_copy(g1, acc.at[dbuf.at[nch - 1]], add=True)
    else:
        pltpu.make_async_copy(table_hbm.at[pl.ds(0, CH)], g0, s0).wait()
        pltpu.sync_copy(g0, acc.at[dbuf.at[nch - 1]], add=True)


def _sc_edge_agg_body(src_hbm, dst_hbm, table_hbm, out_hbm,
                      acc, sidx, didx, g0, g1, s0, s1):
    c = lax.axis_index("c")
    s = lax.axis_index("s")
    w = s * NC + c

    zero16 = jnp.zeros((16,), jnp.float32)

    def zrow(i, carry):
        for j in range(FD // 16):
            g0[i, pl.ds(j * 16, 16)] = zero16
        return carry

    lax.fori_loop(0, CH, zrow, 0)
    base = pl.multiple_of(s * RPT, 8)
    for k in range(RPT // CH):
        pltpu.sync_copy(g0, acc.at[pl.ds(base + k * CH, CH)])
    pltpu.sync_copy(g0.at[pl.ds(0, RPT % CH)],
                    acc.at[pl.ds(base + (RPT // CH) * CH, RPT % CH)])
    plsc.subcore_barrier()

    pltpu.sync_copy(src_hbm.at[w, pl.ds(0, PCH)], sidx)
    pltpu.sync_copy(dst_hbm.at[w, pl.ds(0, PCH)], didx)
    _run_block(table_hbm, acc, sidx, didx, g0, g1, s0, s1, PCH)
    pltpu.sync_copy(src_hbm.at[w, pl.ds(PCH, NCH - PCH)],
                    sidx.at[pl.ds(0, NCH - PCH)])
    pltpu.sync_copy(dst_hbm.at[w, pl.ds(PCH, NCH - PCH)],
                    didx.at[pl.ds(0, NCH - PCH)])
    _run_block(table_hbm, acc, sidx, didx, g0, g1, s0, s1, NCH - PCH)
    plsc.subcore_barrier()

    pltpu.sync_copy(acc.at[pl.ds(base, RPT)], out_hbm.at[c, pl.ds(base, RPT)])


@functools.cache
def _sc_edge_agg():
    mesh = plsc.VectorSubcoreMesh(core_axis_name="c", subcore_axis_name="s",
                                  num_cores=NC, num_subcores=NS)
    return pl.kernel(
        _sc_edge_agg_body,
        out_type=jax.ShapeDtypeStruct((NC, NP, FD), jnp.float32),
        mesh=mesh,
        scratch_types=[
            pltpu.VMEM_SHARED((NP, FD), jnp.float32),
            pltpu.VMEM((PCH, CH), jnp.int32),
            pltpu.VMEM((PCH, CH), jnp.int32),
            pltpu.VMEM((CH, FD), jnp.float32),
            pltpu.VMEM((CH, FD), jnp.float32),
            pltpu.SemaphoreType.DMA,
            pltpu.SemaphoreType.DMA,
        ],
    )


def _sc_deg_body(dst_hbm, out_hbm, acc, didx, obuf, sem):
    c = lax.axis_index("c")
    s = lax.axis_index("s")
    w = s * NC + c

    zero16 = jnp.zeros((16,), jnp.float32)

    def zrow(i, carry):
        obuf[i, pl.ds(0, 16)] = zero16
        return carry

    lax.fori_loop(0, CH, zrow, 0)
    base = pl.multiple_of(s * RPT, 8)
    for k in range(RPT // CH):
        pltpu.sync_copy(obuf, acc.at[pl.ds(base + k * CH, CH)])
    pltpu.sync_copy(obuf.at[pl.ds(0, RPT % CH)],
                    acc.at[pl.ds(base + (RPT // CH) * CH, RPT % CH)])

    one16 = jnp.ones((16,), jnp.float32)

    def orow(i, carry):
        obuf[i, pl.ds(0, 16)] = one16
        return carry

    lax.fori_loop(0, CH, orow, 0)
    pltpu.sync_copy(dst_hbm.at[w], didx)
    plsc.subcore_barrier()

    def chunk(j, carry):
        pltpu.sync_copy(obuf, acc.at[didx.at[j]], add=True)
        return carry

    lax.fori_loop(0, NCH, chunk, 0)
    plsc.subcore_barrier()
    pltpu.sync_copy(acc.at[pl.ds(base, RPT)], out_hbm.at[c, pl.ds(base, RPT)])


@functools.cache
def _sc_deg():
    mesh = plsc.VectorSubcoreMesh(core_axis_name="c", subcore_axis_name="s",
                                  num_cores=NC, num_subcores=NS)
    return pl.kernel(
        _sc_deg_body,
        out_type=jax.ShapeDtypeStruct((NC, NP, 16), jnp.float32),
        mesh=mesh,
        scratch_types=[
            pltpu.VMEM_SHARED((NP, 16), jnp.float32),
            pltpu.VMEM((NCH, CH), jnp.int32),
            pltpu.VMEM((CH, 16), jnp.float32),
            pltpu.SemaphoreType.DMA,
        ],
    )


def _tc_first_body(x_ref, w_ref, d0_ref, d1_ref, scaled_ref, dinv_ref):
    deg = d0_ref[...] + d1_ref[...] + 1.0
    dinv = lax.rsqrt(deg)
    y = jnp.dot(x_ref[...], w_ref[...], preferred_element_type=jnp.float32)
    scaled_ref[...] = dinv * y
    dinv_ref[...] = dinv


def _tc_first(x, w, d0, d1):
    return pl.pallas_call(
        _tc_first_body,
        grid=(NBLK,),
        in_specs=[
            pl.BlockSpec((BR, FD), lambda i: (i, 0)),
            pl.BlockSpec((FD, FD), lambda i: (0, 0)),
            pl.BlockSpec((BR, 1), lambda i: (i, 0)),
            pl.BlockSpec((BR, 1), lambda i: (i, 0)),
        ],
        out_specs=[
            pl.BlockSpec((BR, FD), lambda i: (i, 0)),
            pl.BlockSpec((BR, 1), lambda i: (i, 0)),
        ],
        out_shape=[
            jax.ShapeDtypeStruct((NN, FD), jnp.float32),
            jax.ShapeDtypeStruct((NN, 1), jnp.float32),
        ],
    )(x, w, d0, d1)


def _tc_mid_body(a0_ref, a1_ref, sp_ref, dinv_ref, b_ref, w_ref, out_ref):
    dinv = dinv_ref[...]
    h = dinv * (a0_ref[...] + a1_ref[...] + sp_ref[...]) + b_ref[...]
    h = jnp.maximum(h, 0.0)
    out_ref[...] = dinv * jnp.dot(h, w_ref[...],
                                  preferred_element_type=jnp.float32)


def _tc_mid(a0, a1, sp, dinv, b, w):
    return pl.pallas_call(
        _tc_mid_body,
        grid=(NBLK,),
        in_specs=[
            pl.BlockSpec((BR, FD), lambda i: (i, 0)),
            pl.BlockSpec((BR, FD), lambda i: (i, 0)),
            pl.BlockSpec((BR, FD), lambda i: (i, 0)),
            pl.BlockSpec((BR, 1), lambda i: (i, 0)),
            pl.BlockSpec((1, FD), lambda i: (0, 0)),
            pl.BlockSpec((FD, FD), lambda i: (0, 0)),
        ],
        out_specs=pl.BlockSpec((BR, FD), lambda i: (i, 0)),
        out_shape=jax.ShapeDtypeStruct((NN, FD), jnp.float32),
    )(a0, a1, sp, dinv, b, w)


def _tc_pool_body(a0_ref, a1_ref, sp_ref, dinv_ref, b_ref, batch_ref,
                  wl1_ref, bl1_ref, wl2_ref, bl2_ref, wl3_ref, bl3_ref,
                  out_ref, sum_s, max_s, cnt_s):
    i = pl.program_id(0)

    @pl.when(i == 0)
    def _():
        sum_s[...] = jnp.zeros_like(sum_s)
        cnt_s[...] = jnp.zeros_like(cnt_s)
        max_s[...] = jnp.full_like(max_s, -1e30)

    dinv = dinv_ref[...]
    h = dinv * (a0_ref[...] + a1_ref[...] + sp_ref[...]) + b_ref[...]
    h = jnp.maximum(h, 0.0)

    bvec = batch_ref[...]
    bmin = jnp.min(bvec)
    bmax = jnp.max(bvec)
    for g in range(NG):
        @pl.when(jnp.logical_and(bmin <= g, g <= bmax))
        def _(g=g):
            m = bvec == g
            hm = jnp.where(m, h, 0.0)
            sum_s[g:g + 1, :] = sum_s[g:g + 1, :] + jnp.sum(
                hm, axis=0, keepdims=True)
            cnt = jnp.sum(jnp.where(m, 1.0, 0.0), axis=0, keepdims=True)
            cnt_s[g:g + 1, :] = cnt_s[g:g + 1, :] + cnt
            mx = jnp.max(jnp.where(m, h, -1e30), axis=0, keepdims=True)
            max_s[g:g + 1, :] = jnp.maximum(max_s[g:g + 1, :], mx)

    @pl.when(i == NBLK - 1)
    def _():
        cnt = cnt_s[...]
        sm = sum_s[...]
        mean = sm / jnp.maximum(cnt, 1.0)
        mx = jnp.where(cnt > 0.0, max_s[...], 0.0)
        hg = jnp.concatenate([mean, mx, sm], axis=1)
        z = jnp.dot(hg, wl1_ref[...],
                    preferred_element_type=jnp.float32) + bl1_ref[...]
        z = jnp.maximum(z, 0.0)
        z = jnp.dot(z, wl2_ref[...],
                    preferred_element_type=jnp.float32) + bl2_ref[...]
        z = jnp.maximum(z, 0.0)
        z = jnp.dot(z, wl3_ref[...],
                    preferred_element_type=jnp.float32) + bl3_ref[...]
        out_ref[...] = 1.0 / (1.0 + jnp.exp(-z))


def _tc_pool(a0, a1, sp, dinv, b, batch2d, wl1, bl1, wl2, bl2, wl3p, bl3):
    return pl.pallas_call(
        _tc_pool_body,
        grid=(NBLK,),
        in_specs=[
            pl.BlockSpec((BR, FD), lambda i: (i, 0)),
            pl.BlockSpec((BR, FD), lambda i: (i, 0)),
            pl.BlockSpec((BR, FD), lambda i: (i, 0)),
            pl.BlockSpec((BR, 1), lambda i: (i, 0)),
            pl.BlockSpec((1, FD), lambda i: (0, 0)),
            pl.BlockSpec((BR, 1), lambda i: (i, 0)),
            pl.BlockSpec((3 * FD, 3 * FD), lambda i: (0, 0)),
            pl.BlockSpec((1, 3 * FD), lambda i: (0, 0)),
            pl.BlockSpec((3 * FD, FD), lambda i: (0, 0)),
            pl.BlockSpec((1, FD), lambda i: (0, 0)),
            pl.BlockSpec((FD, FD), lambda i: (0, 0)),
            pl.BlockSpec((1, 1), lambda i: (0, 0)),
        ],
        out_specs=pl.BlockSpec((NG, FD), lambda i: (0, 0)),
        out_shape=jax.ShapeDtypeStruct((NG, FD), jnp.float32),
        scratch_shapes=[
            pltpu.VMEM((NG, FD), jnp.float32),
            pltpu.VMEM((NG, FD), jnp.float32),
            pltpu.VMEM((NG, FD), jnp.float32),
        ],
    )(a0, a1, sp, dinv, b, batch2d, wl1, bl1, wl2, bl2, wl3p, bl3)


def kernel(x, edge_index, batch, Wg0, bg0, Wg1, bg1,
           Wl1, bl1, Wl2, bl2, Wl3, bl3):
    src = edge_index[0]
    dst = edge_index[1]
    npad = EPAD - EPW
    padd = NN + (jnp.arange(npad, dtype=jnp.int32) % (NP - NN))
    pads = jnp.arange(npad, dtype=jnp.int32) % NN
    srcp = jnp.concatenate(
        [src.reshape(NW, EPW),
         jnp.broadcast_to(pads, (NW, npad))], axis=1).reshape(NW, NCH, CH)
    dstp = jnp.concatenate(
        [dst.reshape(NW, EPW),
         jnp.broadcast_to(padd, (NW, npad))], axis=1).reshape(NW, NCH, CH)

    degraw = _sc_deg()(dstp)
    d0 = degraw[0, :, 0:1]
    d1 = degraw[1, :, 0:1]

    scaled0, dinv = _tc_first(x, Wg0, d0, d1)
    agg = _sc_edge_agg()(srcp, dstp, scaled0)
    scaled1 = _tc_mid(agg[0], agg[1], scaled0, dinv, bg0.reshape(1, FD), Wg1)
    agg = _sc_edge_agg()(srcp, dstp, scaled1)
    scaled2 = _tc_mid(agg[0], agg[1], scaled1, dinv, bg1.reshape(1, FD), Wg1)
    agg = _sc_edge_agg()(srcp, dstp, scaled2)

    wl3p = jnp.pad(Wl3, ((0, 0), (0, FD - 1)))
    outp = _tc_pool(agg[0], agg[1], scaled2, dinv, bg1.reshape(1, FD),
                    batch.reshape(NN, 1).astype(jnp.int32),
                    Wl1, bl1.reshape(1, 3 * FD), Wl2, bl2.reshape(1, FD),
                    wl3p, bl3.reshape(1, 1))
    return outp[:, 0:1]
